# baseline (speedup 1.0000x reference)
import functools

import jax
import jax.numpy as jnp
from jax import lax
from jax.experimental import pallas as pl
from jax.experimental.pallas import tpu as pltpu
from jax.experimental.pallas import tpu_sc as plsc

N = 2048
E = 32768
F = 128
LANES = 16
NC = 2
NS = 16
EPS = E // NS
GROUPS = EPS // LANES
ROWS_PER_SC = 512
NPASS = N // (NC * ROWS_PER_SC)
SPROWS = ROWS_PER_SC * (N // LANES)
SHARE = SPROWS // NS
NCHUNK = N // LANES
CHE = 1024
SCAT = CHE // 128
JUNK = 128
DEGROWS = ROWS_PER_SC // LANES
DEGBASE = SPROWS + JUNK


def _build_lcount(row, col):
  mesh = plsc.VectorSubcoreMesh(core_axis_name="c", subcore_axis_name="s")

  @functools.partial(
      pl.kernel,
      out_type=[
          jax.ShapeDtypeStruct((N * NCHUNK, LANES), jnp.float32),
          jax.ShapeDtypeStruct((N // LANES, LANES), jnp.float32),
      ],
      mesh=mesh,
      compiler_params=pltpu.CompilerParams(needs_layout_passes=False,
                                           use_tc_tiling_on_sc=False),
      scratch_types=[
          pltpu.VMEM((EPS,), jnp.int32),
          pltpu.VMEM((EPS,), jnp.int32),
          pltpu.VMEM((CHE, LANES), jnp.float32),
          pltpu.VMEM((CHE, LANES), jnp.float32),
          pltpu.VMEM((SCAT, 128), jnp.int32),
          pltpu.VMEM((SCAT, 128), jnp.int32),
          pltpu.VMEM((512, LANES), jnp.float32),
          pltpu.VMEM_SHARED((SPROWS + JUNK + DEGROWS, LANES), jnp.float32),
      ],
  )
  def build(row_hbm, col_hbm, out_hbm, degp_hbm, rowv, colv, oh, oh2, idxb,
            idx2, zb, slab):
    c = lax.axis_index("c")
    s = lax.axis_index("s")
    pltpu.sync_copy(row_hbm.at[pl.ds(s * EPS, EPS)], rowv)
    pltpu.sync_copy(col_hbm.at[pl.ds(s * EPS, EPS)], colv)

    zeros16 = jnp.zeros((LANES,), jnp.float32)
    ones16 = jnp.ones((LANES,), jnp.float32)
    lanes_iota = lax.iota(jnp.int32, LANES)

    def _zerozb(i, _):
      zb[i, :] = zeros16
      return 0
    lax.fori_loop(0, 512, _zerozb, 0)

    for p in range(NPASS):
      lo = (p * NC + c) * ROWS_PER_SC

      for q in range(SHARE // 512):
        pltpu.sync_copy(zb, slab.at[pl.ds(s * SHARE + q * 512, 512)])

      @pl.when(s == 0)
      def _():
        pltpu.sync_copy(zb.at[pl.ds(0, DEGROWS)],
                        slab.at[pl.ds(DEGBASE, DEGROWS)])

      plsc.subcore_barrier()

      for ch in range(EPS // CHE):
        def _zero(i, _):
          oh[i, :] = zeros16
          oh2[i, :] = zeros16
          return 0
        lax.fori_loop(0, CHE, _zero, 0)

        def _fill(g, _):
          e0 = ch * CHE + g * LANES
          r = rowv[pl.ds(e0, LANES)]
          cc = colv[pl.ds(e0, LANES)]
          m = (r >= lo) & (r < lo + ROWS_PER_SC) & (r != cc)
          rl = r - lo
          flat = rl * NCHUNK + lax.shift_right_logical(cc, 2 + 2)
          junk = SPROWS + lanes_iota + ((g % 8) * LANES)
          idxb[g // 8, pl.ds((g % 8) * LANES, LANES)] = jnp.where(m, flat,
                                                                  junk)
          flat2 = DEGBASE + lax.shift_right_logical(rl, 2 + 2)
          idx2[g // 8, pl.ds((g % 8) * LANES, LANES)] = jnp.where(m, flat2,
                                                                  junk)
          slot = lanes_iota + g * LANES
          plsc.store_scatter(oh, [slot, lax.bitwise_and(cc, LANES - 1)],
                             ones16)
          plsc.store_scatter(oh2, [slot, lax.bitwise_and(rl, LANES - 1)],
                             ones16)
          return 0
        lax.fori_loop(0, CHE // LANES, _fill, 0)

        for j in range(SCAT):
          pltpu.sync_copy(oh.at[pl.ds(j * 128, 128)], slab.at[idxb.at[j]],
                          add=True)
          pltpu.sync_copy(oh2.at[pl.ds(j * 128, 128)], slab.at[idx2.at[j]],
                          add=True)

      plsc.subcore_barrier()
      obase = lo * NCHUNK + s * SHARE
      pltpu.sync_copy(slab.at[pl.ds(s * SHARE, SHARE)],
                      out_hbm.at[pl.ds(obase, SHARE)])

      @pl.when(s == 0)
      def _():
        pltpu.sync_copy(slab.at[pl.ds(DEGBASE, DEGROWS)],
                        degp_hbm.at[pl.ds((p * NC + c) * DEGROWS, DEGROWS)])

  return build(row, col)


TR = 512
TGRID = N // TR
B = 4


def _k0_body(sa, x, t0):
  ii = lax.broadcasted_iota(jnp.int32, (TR, TR), 0)
  jj = lax.broadcasted_iota(jnp.int32, (TR, TR), 1)
  for b in range(B):
    d = jnp.sum(jnp.where(ii == jj, sa[b], 0.0), axis=1, keepdims=True)
    t0[b] = d * x[b]


def _dis_of(deg):
  pos = deg > 0.0
  return jnp.where(pos, lax.rsqrt(jnp.where(pos, deg, 1.0)), 0.0)


def _k1_body(sa, lc, deg, degt, t0, t1):
  disc = _dis_of(deg[:, :])
  dist = _dis_of(degt[:, :])
  scale = -dist / (degt[:, :] + 2.0)
  lcd = lc[:, :] * disc
  for b in range(B):
    acc = jnp.dot(sa[b] * lcd, t0[b], preferred_element_type=jnp.float32)
    t1[b] = scale * acc


def _k2_body(lc, deg, degt, t1f, t0t, t1t, w, bias, out):
  disc = _dis_of(deg[:, :])
  dist = _dis_of(degt[:, :])
  scale = -2.0 * dist / (degt[:, :] + 2.0)
  mc = lc[:, :] * disc
  for b in range(B):
    prop = scale * jnp.dot(mc, t1f[b], preferred_element_type=jnp.float32)
    t2 = prop - t0t[b]
    out[b] = (jnp.dot(t0t[b], w[0], preferred_element_type=jnp.float32)
              + jnp.dot(t1t[b], w[1], preferred_element_type=jnp.float32)
              + jnp.dot(t2, w[2], preferred_element_type=jnp.float32)
              + bias[:, :])


def kernel(x, edge_index, spatial_attention, weight, bias):
  row = edge_index[0]
  col = edge_index[1]
  lc16, degp = _build_lcount(row, col)
  lc = lc16.reshape(N, N)

  t0 = pl.pallas_call(
      _k0_body,
      grid=(TGRID,),
      in_specs=[
          pl.BlockSpec((B, TR, TR), lambda t: (0, t, t)),
          pl.BlockSpec((B, TR, F), lambda t: (0, t, 0)),
      ],
      out_specs=pl.BlockSpec((B, TR, F), lambda t: (0, t, 0)),
      out_shape=jax.ShapeDtypeStruct((B, N, F), jnp.float32),
  )(spatial_attention, x)

  deg = degp.reshape(1, N)
  degt = degp.reshape(N, 1)

  t1 = pl.pallas_call(
      _k1_body,
      grid=(TGRID,),
      in_specs=[
          pl.BlockSpec((B, TR, N), lambda t: (0, t, 0)),
          pl.BlockSpec((TR, N), lambda t: (t, 0)),
          pl.BlockSpec((1, N), lambda t: (0, 0)),
          pl.BlockSpec((TR, 1), lambda t: (t, 0)),
          pl.BlockSpec((B, N, F), lambda t: (0, 0, 0)),
      ],
      out_specs=pl.BlockSpec((B, TR, F), lambda t: (0, t, 0)),
      out_shape=jax.ShapeDtypeStruct((B, N, F), jnp.float32),
  )(spatial_attention, lc, deg, degt, t0)

  out = pl.pallas_call(
      _k2_body,
      grid=(TGRID,),
      in_specs=[
          pl.BlockSpec((TR, N), lambda t: (t, 0)),
          pl.BlockSpec((1, N), lambda t: (0, 0)),
          pl.BlockSpec((TR, 1), lambda t: (t, 0)),
          pl.BlockSpec((B, N, F), lambda t: (0, 0, 0)),
          pl.BlockSpec((B, TR, F), lambda t: (0, t, 0)),
          pl.BlockSpec((B, TR, F), lambda t: (0, t, 0)),
          pl.BlockSpec((3, F, F), lambda t: (0, 0, 0)),
          pl.BlockSpec((1, F), lambda t: (0, 0)),
      ],
      out_specs=pl.BlockSpec((B, TR, F), lambda t: (0, t, 0)),
      out_shape=jax.ShapeDtypeStruct((B, N, F), jnp.float32),
  )(lc, deg, degt, t1, t0, t1, weight, bias.reshape(1, F))

  return out


# speedup vs baseline: 1.0084x; 1.0084x over previous
import functools

import jax
import jax.numpy as jnp
from jax import lax
from jax.experimental import pallas as pl
from jax.experimental.pallas import tpu as pltpu
from jax.experimental.pallas import tpu_sc as plsc

N = 2048
E = 32768
F = 128
LANES = 16
NC = 2
NS = 16
EPS = E // NS
GROUPS = EPS // LANES
ROWS_PER_SC = 512
NPASS = N // (NC * ROWS_PER_SC)
SPROWS = ROWS_PER_SC * (N // LANES)
SHARE = SPROWS // NS
NCHUNK = N // LANES
CHE = 1024
SCAT = CHE // 128
JUNK = 128
DEGROWS = ROWS_PER_SC // LANES
DEGBASE = SPROWS + JUNK


def _build_lcount(row, col):
  mesh = plsc.VectorSubcoreMesh(core_axis_name="c", subcore_axis_name="s")

  @functools.partial(
      pl.kernel,
      out_type=[
          jax.ShapeDtypeStruct((N * NCHUNK, LANES), jnp.float32),
          jax.ShapeDtypeStruct((N // LANES, LANES), jnp.float32),
      ],
      mesh=mesh,
      compiler_params=pltpu.CompilerParams(needs_layout_passes=False,
                                           use_tc_tiling_on_sc=False),
      scratch_types=[
          pltpu.VMEM((EPS,), jnp.int32),
          pltpu.VMEM((EPS,), jnp.int32),
          pltpu.VMEM((CHE, LANES), jnp.float32),
          pltpu.VMEM((CHE, LANES), jnp.float32),
          pltpu.VMEM((SCAT, 128), jnp.int32),
          pltpu.VMEM((SCAT, 128), jnp.int32),
          pltpu.VMEM((512, LANES), jnp.float32),
          pltpu.VMEM_SHARED((SPROWS + JUNK + DEGROWS, LANES), jnp.float32),
      ],
  )
  def build(row_hbm, col_hbm, out_hbm, degp_hbm, rowv, colv, oh, oh2, idxb,
            idx2, zb, slab):
    c = lax.axis_index("c")
    s = lax.axis_index("s")
    pltpu.sync_copy(row_hbm.at[pl.ds(s * EPS, EPS)], rowv)
    pltpu.sync_copy(col_hbm.at[pl.ds(s * EPS, EPS)], colv)

    zeros16 = jnp.zeros((LANES,), jnp.float32)
    ones16 = jnp.ones((LANES,), jnp.float32)
    lanes_iota = lax.iota(jnp.int32, LANES)

    def _zerozb(i, _):
      zb[i, :] = zeros16
      return 0
    lax.fori_loop(0, 512, _zerozb, 0)

    for p in range(NPASS):
      lo = (p * NC + c) * ROWS_PER_SC

      for q in range(SHARE // 512):
        pltpu.sync_copy(zb, slab.at[pl.ds(s * SHARE + q * 512, 512)])

      @pl.when(s == 0)
      def _():
        pltpu.sync_copy(zb.at[pl.ds(0, DEGROWS)],
                        slab.at[pl.ds(DEGBASE, DEGROWS)])

      plsc.subcore_barrier()

      for ch in range(EPS // CHE):
        def _zero(i, _):
          oh[i, :] = zeros16
          oh2[i, :] = zeros16
          return 0
        lax.fori_loop(0, CHE, _zero, 0)

        def _fill(g, _):
          e0 = ch * CHE + g * LANES
          r = rowv[pl.ds(e0, LANES)]
          cc = colv[pl.ds(e0, LANES)]
          m = (r >= lo) & (r < lo + ROWS_PER_SC) & (r != cc)
          rl = r - lo
          flat = rl * NCHUNK + lax.shift_right_logical(cc, 2 + 2)
          junk = SPROWS + lanes_iota + ((g % 8) * LANES)
          idxb[g // 8, pl.ds((g % 8) * LANES, LANES)] = jnp.where(m, flat,
                                                                  junk)
          flat2 = DEGBASE + lax.shift_right_logical(rl, 2 + 2)
          idx2[g // 8, pl.ds((g % 8) * LANES, LANES)] = jnp.where(m, flat2,
                                                                  junk)
          slot = lanes_iota + g * LANES
          plsc.store_scatter(oh, [slot, lax.bitwise_and(cc, LANES - 1)],
                             ones16)
          plsc.store_scatter(oh2, [slot, lax.bitwise_and(rl, LANES - 1)],
                             ones16)
          return 0
        lax.fori_loop(0, CHE // LANES, _fill, 0)

        for j in range(SCAT):
          pltpu.sync_copy(oh.at[pl.ds(j * 128, 128)], slab.at[idxb.at[j]],
                          add=True)
          pltpu.sync_copy(oh2.at[pl.ds(j * 128, 128)], slab.at[idx2.at[j]],
                          add=True)

      plsc.subcore_barrier()
      obase = lo * NCHUNK + s * SHARE
      pltpu.sync_copy(slab.at[pl.ds(s * SHARE, SHARE)],
                      out_hbm.at[pl.ds(obase, SHARE)])

      @pl.when(s == 0)
      def _():
        pltpu.sync_copy(slab.at[pl.ds(DEGBASE, DEGROWS)],
                        degp_hbm.at[pl.ds((p * NC + c) * DEGROWS, DEGROWS)])

  return build(row, col)


TR = 128
TGRID = N // TR
B = 4


def _k0_body(sa, x, t0):
  ii = lax.broadcasted_iota(jnp.int32, (TR, TR), 0)
  jj = lax.broadcasted_iota(jnp.int32, (TR, TR), 1)
  for b in range(B):
    d = jnp.sum(jnp.where(ii == jj, sa[b], 0.0), axis=1, keepdims=True)
    t0[b] = d * x[b]


def _dis_of(deg):
  pos = deg > 0.0
  return jnp.where(pos, lax.rsqrt(jnp.where(pos, deg, 1.0)), 0.0)


def _k1_body(sa, lc, deg, degt, t0, t1):
  disc = _dis_of(deg[:, :])
  dist = _dis_of(degt[:, :])
  scale = -dist / (degt[:, :] + 2.0)
  lcd = lc[:, :] * disc
  for b in range(B):
    acc = jnp.dot(sa[b] * lcd, t0[b], preferred_element_type=jnp.float32)
    t1[b] = scale * acc


def _k2_body(lc, deg, degt, t1f, t0t, t1t, w, bias, out):
  disc = _dis_of(deg[:, :])
  dist = _dis_of(degt[:, :])
  scale = -2.0 * dist / (degt[:, :] + 2.0)
  mc = lc[:, :] * disc
  for b in range(B):
    prop = scale * jnp.dot(mc, t1f[b], preferred_element_type=jnp.float32)
    t2 = prop - t0t[b]
    out[b] = (jnp.dot(t0t[b], w[0], preferred_element_type=jnp.float32)
              + jnp.dot(t1t[b], w[1], preferred_element_type=jnp.float32)
              + jnp.dot(t2, w[2], preferred_element_type=jnp.float32)
              + bias[:, :])


def kernel(x, edge_index, spatial_attention, weight, bias):
  row = edge_index[0]
  col = edge_index[1]
  lc16, degp = _build_lcount(row, col)
  lc = lc16.reshape(N, N)

  t0 = pl.pallas_call(
      _k0_body,
      grid=(TGRID,),
      in_specs=[
          pl.BlockSpec((B, TR, TR), lambda t: (0, t, t)),
          pl.BlockSpec((B, TR, F), lambda t: (0, t, 0)),
      ],
      out_specs=pl.BlockSpec((B, TR, F), lambda t: (0, t, 0)),
      out_shape=jax.ShapeDtypeStruct((B, N, F), jnp.float32),
  )(spatial_attention, x)

  deg = degp.reshape(1, N)
  degt = degp.reshape(N, 1)

  t1 = pl.pallas_call(
      _k1_body,
      grid=(TGRID,),
      in_specs=[
          pl.BlockSpec((B, TR, N), lambda t: (0, t, 0)),
          pl.BlockSpec((TR, N), lambda t: (t, 0)),
          pl.BlockSpec((1, N), lambda t: (0, 0)),
          pl.BlockSpec((TR, 1), lambda t: (t, 0)),
          pl.BlockSpec((B, N, F), lambda t: (0, 0, 0)),
      ],
      out_specs=pl.BlockSpec((B, TR, F), lambda t: (0, t, 0)),
      out_shape=jax.ShapeDtypeStruct((B, N, F), jnp.float32),
  )(spatial_attention, lc, deg, degt, t0)

  out = pl.pallas_call(
      _k2_body,
      grid=(TGRID,),
      in_specs=[
          pl.BlockSpec((TR, N), lambda t: (t, 0)),
          pl.BlockSpec((1, N), lambda t: (0, 0)),
          pl.BlockSpec((TR, 1), lambda t: (t, 0)),
          pl.BlockSpec((B, N, F), lambda t: (0, 0, 0)),
          pl.BlockSpec((B, TR, F), lambda t: (0, t, 0)),
          pl.BlockSpec((B, TR, F), lambda t: (0, t, 0)),
          pl.BlockSpec((3, F, F), lambda t: (0, 0, 0)),
          pl.BlockSpec((1, F), lambda t: (0, 0)),
      ],
      out_specs=pl.BlockSpec((B, TR, F), lambda t: (0, t, 0)),
      out_shape=jax.ShapeDtypeStruct((B, N, F), jnp.float32),
  )(lc, deg, degt, t1, t0, t1, weight, bias.reshape(1, F))

  return out


# speedup vs baseline: 1.0444x; 1.0358x over previous
import functools

import jax
import jax.numpy as jnp
from jax import lax
from jax.experimental import pallas as pl
from jax.experimental.pallas import tpu as pltpu
from jax.experimental.pallas import tpu_sc as plsc

N = 2048
E = 32768
F = 128
LANES = 16
NC = 2
NS = 16
EPS = E // NS
GROUPS = EPS // LANES
ROWS_PER_SC = 512
NPASS = N // (NC * ROWS_PER_SC)
SPROWS = ROWS_PER_SC * (N // LANES)
SHARE = SPROWS // NS
NCHUNK = N // LANES
CHE = 1024
SCAT = CHE // 128
JUNK = 128
DEGROWS = ROWS_PER_SC // LANES
DEGBASE = SPROWS + JUNK


def _build_lcount(row, col):
  mesh = plsc.VectorSubcoreMesh(core_axis_name="c", subcore_axis_name="s")

  @functools.partial(
      pl.kernel,
      out_type=[
          jax.ShapeDtypeStruct((N * NCHUNK, LANES), jnp.float32),
          jax.ShapeDtypeStruct((N // LANES, LANES), jnp.float32),
      ],
      mesh=mesh,
      compiler_params=pltpu.CompilerParams(needs_layout_passes=False,
                                           use_tc_tiling_on_sc=False),
      scratch_types=[
          pltpu.VMEM((EPS,), jnp.int32),
          pltpu.VMEM((EPS,), jnp.int32),
          pltpu.VMEM((CHE, LANES), jnp.float32),
          pltpu.VMEM((CHE, LANES), jnp.float32),
          pltpu.VMEM((SCAT, 128), jnp.int32),
          pltpu.VMEM((SCAT, 128), jnp.int32),
          pltpu.VMEM((512, LANES), jnp.float32),
          pltpu.VMEM_SHARED((SPROWS + JUNK + DEGROWS, LANES), jnp.float32),
          pltpu.SemaphoreType.DMA,
      ],
  )
  def build(row_hbm, col_hbm, out_hbm, degp_hbm, rowv, colv, oh, oh2, idxb,
            idx2, zb, slab, sem):
    c = lax.axis_index("c")
    s = lax.axis_index("s")
    pltpu.sync_copy(row_hbm.at[pl.ds(s * EPS, EPS)], rowv)
    pltpu.sync_copy(col_hbm.at[pl.ds(s * EPS, EPS)], colv)

    zeros16 = jnp.zeros((LANES,), jnp.float32)
    ones16 = jnp.ones((LANES,), jnp.float32)
    lanes_iota = lax.iota(jnp.int32, LANES)

    def _zerozb(i, _):
      zb[i, :] = zeros16
      return 0
    lax.fori_loop(0, 512, _zerozb, 0)

    for p in range(NPASS):
      lo = (p * NC + c) * ROWS_PER_SC

      zdesc = [pltpu.async_copy(zb, slab.at[pl.ds(s * SHARE + q * 512, 512)],
                                sem)
               for q in range(SHARE // 512)]
      for d in zdesc:
        d.wait()

      @pl.when(s == 0)
      def _():
        pltpu.sync_copy(zb.at[pl.ds(0, DEGROWS)],
                        slab.at[pl.ds(DEGBASE, DEGROWS)])

      plsc.subcore_barrier()

      for ch in range(EPS // CHE):
        def _zero(i, _):
          oh[i, :] = zeros16
          oh2[i, :] = zeros16
          return 0
        lax.fori_loop(0, CHE, _zero, 0)

        def _fill(g, _):
          e0 = ch * CHE + g * LANES
          r = rowv[pl.ds(e0, LANES)]
          cc = colv[pl.ds(e0, LANES)]
          m = (r >= lo) & (r < lo + ROWS_PER_SC) & (r != cc)
          rl = r - lo
          flat = rl * NCHUNK + lax.shift_right_logical(cc, 2 + 2)
          junk = SPROWS + lanes_iota + ((g % 8) * LANES)
          idxb[g // 8, pl.ds((g % 8) * LANES, LANES)] = jnp.where(m, flat,
                                                                  junk)
          flat2 = DEGBASE + lax.shift_right_logical(rl, 2 + 2)
          idx2[g // 8, pl.ds((g % 8) * LANES, LANES)] = jnp.where(m, flat2,
                                                                  junk)
          slot = lanes_iota + g * LANES
          plsc.store_scatter(oh, [slot, lax.bitwise_and(cc, LANES - 1)],
                             ones16)
          plsc.store_scatter(oh2, [slot, lax.bitwise_and(rl, LANES - 1)],
                             ones16)
          return 0
        lax.fori_loop(0, CHE // LANES, _fill, 0)

        descs = []
        for j in range(SCAT):
          descs.append(pltpu.async_copy(oh.at[pl.ds(j * 128, 128)],
                                        slab.at[idxb.at[j]], sem, add=True))
          descs.append(pltpu.async_copy(oh2.at[pl.ds(j * 128, 128)],
                                        slab.at[idx2.at[j]], sem, add=True))
        for d in descs:
          d.wait()

      plsc.subcore_barrier()
      obase = lo * NCHUNK + s * SHARE
      pltpu.sync_copy(slab.at[pl.ds(s * SHARE, SHARE)],
                      out_hbm.at[pl.ds(obase, SHARE)])

      @pl.when(s == 0)
      def _():
        pltpu.sync_copy(slab.at[pl.ds(DEGBASE, DEGROWS)],
                        degp_hbm.at[pl.ds((p * NC + c) * DEGROWS, DEGROWS)])

  return build(row, col)


TR = 256
TGRID = N // TR
B = 4


def _k0_body(sa, x, t0):
  ii = lax.broadcasted_iota(jnp.int32, (TR, TR), 0)
  jj = lax.broadcasted_iota(jnp.int32, (TR, TR), 1)
  for b in range(B):
    d = jnp.sum(jnp.where(ii == jj, sa[b], 0.0), axis=1, keepdims=True)
    t0[b] = d * x[b]


def _dis_of(deg):
  pos = deg > 0.0
  return jnp.where(pos, lax.rsqrt(jnp.where(pos, deg, 1.0)), 0.0)


def _k1_body(sa, lc, deg, degt, t0, t1):
  disc = _dis_of(deg[:, :])
  dist = _dis_of(degt[:, :])
  scale = -dist / (degt[:, :] + 2.0)
  lcd = lc[:, :] * disc
  for b in range(B):
    acc = jnp.dot(sa[b] * lcd, t0[b], preferred_element_type=jnp.float32)
    t1[b] = scale * acc


def _k2_body(lc, deg, degt, t1f, t0t, t1t, w, bias, out):
  disc = _dis_of(deg[:, :])
  dist = _dis_of(degt[:, :])
  scale = -2.0 * dist / (degt[:, :] + 2.0)
  mc = lc[:, :] * disc
  for b in range(B):
    prop = scale * jnp.dot(mc, t1f[b], preferred_element_type=jnp.float32)
    t2 = prop - t0t[b]
    out[b] = (jnp.dot(t0t[b], w[0], preferred_element_type=jnp.float32)
              + jnp.dot(t1t[b], w[1], preferred_element_type=jnp.float32)
              + jnp.dot(t2, w[2], preferred_element_type=jnp.float32)
              + bias[:, :])


def kernel(x, edge_index, spatial_attention, weight, bias):
  row = edge_index[0]
  col = edge_index[1]
  lc16, degp = _build_lcount(row, col)
  lc = lc16.reshape(N, N)

  t0 = pl.pallas_call(
      _k0_body,
      grid=(TGRID,),
      in_specs=[
          pl.BlockSpec((B, TR, TR), lambda t: (0, t, t)),
          pl.BlockSpec((B, TR, F), lambda t: (0, t, 0)),
      ],
      out_specs=pl.BlockSpec((B, TR, F), lambda t: (0, t, 0)),
      out_shape=jax.ShapeDtypeStruct((B, N, F), jnp.float32),
  )(spatial_attention, x)

  deg = degp.reshape(1, N)
  degt = degp.reshape(N, 1)

  t1 = pl.pallas_call(
      _k1_body,
      grid=(TGRID,),
      in_specs=[
          pl.BlockSpec((B, TR, N), lambda t: (0, t, 0)),
          pl.BlockSpec((TR, N), lambda t: (t, 0)),
          pl.BlockSpec((1, N), lambda t: (0, 0)),
          pl.BlockSpec((TR, 1), lambda t: (t, 0)),
          pl.BlockSpec((B, N, F), lambda t: (0, 0, 0)),
      ],
      out_specs=pl.BlockSpec((B, TR, F), lambda t: (0, t, 0)),
      out_shape=jax.ShapeDtypeStruct((B, N, F), jnp.float32),
  )(spatial_attention, lc, deg, degt, t0)

  out = pl.pallas_call(
      _k2_body,
      grid=(TGRID,),
      in_specs=[
          pl.BlockSpec((TR, N), lambda t: (t, 0)),
          pl.BlockSpec((1, N), lambda t: (0, 0)),
          pl.BlockSpec((TR, 1), lambda t: (t, 0)),
          pl.BlockSpec((B, N, F), lambda t: (0, 0, 0)),
          pl.BlockSpec((B, TR, F), lambda t: (0, t, 0)),
          pl.BlockSpec((B, TR, F), lambda t: (0, t, 0)),
          pl.BlockSpec((3, F, F), lambda t: (0, 0, 0)),
          pl.BlockSpec((1, F), lambda t: (0, 0)),
      ],
      out_specs=pl.BlockSpec((B, TR, F), lambda t: (0, t, 0)),
      out_shape=jax.ShapeDtypeStruct((B, N, F), jnp.float32),
  )(lc, deg, degt, t1, t0, t1, weight, bias.reshape(1, F))

  return out


# speedup vs baseline: 1.0735x; 1.0278x over previous
import functools

import jax
import jax.numpy as jnp
from jax import lax
from jax.experimental import pallas as pl
from jax.experimental.pallas import tpu as pltpu
from jax.experimental.pallas import tpu_sc as plsc

N = 2048
E = 32768
F = 128
LANES = 16
NC = 2
NS = 16
EPS = E // NS
GROUPS = EPS // LANES
ROWS_PER_SC = 512
NPASS = N // (NC * ROWS_PER_SC)
SPROWS = ROWS_PER_SC * (N // LANES)
SHARE = SPROWS // NS
NCHUNK = N // LANES
CHE = 512
SCAT = CHE // 128
JUNK = 128
DEGROWS = ROWS_PER_SC // LANES
DEGBASE = SPROWS + JUNK


def _build_lcount(row, col):
  mesh = plsc.VectorSubcoreMesh(core_axis_name="c", subcore_axis_name="s")

  @functools.partial(
      pl.kernel,
      out_type=[
          jax.ShapeDtypeStruct((N * NCHUNK, LANES), jnp.float32),
          jax.ShapeDtypeStruct((N // LANES, LANES), jnp.float32),
      ],
      mesh=mesh,
      compiler_params=pltpu.CompilerParams(needs_layout_passes=False,
                                           use_tc_tiling_on_sc=False),
      scratch_types=[
          pltpu.VMEM((EPS,), jnp.int32),
          pltpu.VMEM((EPS,), jnp.int32),
          pltpu.VMEM((2, CHE, LANES), jnp.float32),
          pltpu.VMEM((2, CHE, LANES), jnp.float32),
          pltpu.VMEM((2, SCAT, 128), jnp.int32),
          pltpu.VMEM((2, SCAT, 128), jnp.int32),
          pltpu.VMEM((512, LANES), jnp.float32),
          pltpu.VMEM_SHARED((SPROWS + JUNK + DEGROWS, LANES), jnp.float32),
          pltpu.SemaphoreType.DMA,
      ],
  )
  def build(row_hbm, col_hbm, out_hbm, degp_hbm, rowv, colv, oh, oh2, idxb,
            idx2, zb, slab, sem):
    c = lax.axis_index("c")
    s = lax.axis_index("s")
    pltpu.sync_copy(row_hbm.at[pl.ds(s * EPS, EPS)], rowv)
    pltpu.sync_copy(col_hbm.at[pl.ds(s * EPS, EPS)], colv)

    zeros16 = jnp.zeros((LANES,), jnp.float32)
    ones16 = jnp.ones((LANES,), jnp.float32)
    lanes_iota = lax.iota(jnp.int32, LANES)

    def _zerozb(i, _):
      zb[i, :] = zeros16
      return 0
    lax.fori_loop(0, 512, _zerozb, 0)

    for p in range(NPASS):
      lo = (p * NC + c) * ROWS_PER_SC

      zdesc = [pltpu.async_copy(zb, slab.at[pl.ds(s * SHARE + q * 512, 512)],
                                sem)
               for q in range(SHARE // 512)]
      for d in zdesc:
        d.wait()

      @pl.when(s == 0)
      def _():
        pltpu.sync_copy(zb.at[pl.ds(0, DEGROWS)],
                        slab.at[pl.ds(DEGBASE, DEGROWS)])

      plsc.subcore_barrier()

      pending = [[], []]
      for ch in range(EPS // CHE):
        hb = ch % 2
        for d in pending[hb]:
          d.wait()
        pending[hb] = []

        def _zero(i, _):
          oh[hb, i, :] = zeros16
          oh2[hb, i, :] = zeros16
          return 0
        lax.fori_loop(0, CHE, _zero, 0)

        def _fill(g, _):
          e0 = ch * CHE + g * LANES
          r = rowv[pl.ds(e0, LANES)]
          cc = colv[pl.ds(e0, LANES)]
          m = (r >= lo) & (r < lo + ROWS_PER_SC) & (r != cc)
          rl = r - lo
          flat = rl * NCHUNK + lax.shift_right_logical(cc, 2 + 2)
          junk = SPROWS + lanes_iota + ((g % 8) * LANES)
          idxb[hb, g // 8, pl.ds((g % 8) * LANES, LANES)] = jnp.where(m, flat,
                                                                     junk)
          flat2 = DEGBASE + lax.shift_right_logical(rl, 2 + 2)
          idx2[hb, g // 8, pl.ds((g % 8) * LANES, LANES)] = jnp.where(m, flat2,
                                                                     junk)
          slot = lanes_iota + g * LANES
          plsc.store_scatter(oh.at[hb], [slot, lax.bitwise_and(cc, LANES - 1)],
                             ones16)
          plsc.store_scatter(oh2.at[hb],
                             [slot, lax.bitwise_and(rl, LANES - 1)], ones16)
          return 0
        lax.fori_loop(0, CHE // LANES, _fill, 0)

        for j in range(SCAT):
          pending[hb].append(
              pltpu.async_copy(oh.at[hb, pl.ds(j * 128, 128)],
                               slab.at[idxb.at[hb, j]], sem, add=True))
          pending[hb].append(
              pltpu.async_copy(oh2.at[hb, pl.ds(j * 128, 128)],
                               slab.at[idx2.at[hb, j]], sem, add=True))

      for lst in pending:
        for d in lst:
          d.wait()

      plsc.subcore_barrier()
      obase = lo * NCHUNK + s * SHARE
      pltpu.sync_copy(slab.at[pl.ds(s * SHARE, SHARE)],
                      out_hbm.at[pl.ds(obase, SHARE)])

      @pl.when(s == 0)
      def _():
        pltpu.sync_copy(slab.at[pl.ds(DEGBASE, DEGROWS)],
                        degp_hbm.at[pl.ds((p * NC + c) * DEGROWS, DEGROWS)])

  return build(row, col)


TR = 256
TGRID = N // TR
B = 4


def _k0_body(sa, x, t0):
  ii = lax.broadcasted_iota(jnp.int32, (TR, TR), 0)
  jj = lax.broadcasted_iota(jnp.int32, (TR, TR), 1)
  for b in range(B):
    d = jnp.sum(jnp.where(ii == jj, sa[b], 0.0), axis=1, keepdims=True)
    t0[b] = d * x[b]


def _dis_of(deg):
  pos = deg > 0.0
  return jnp.where(pos, lax.rsqrt(jnp.where(pos, deg, 1.0)), 0.0)


def _k1_body(sa, lc, deg, degt, t0, t1):
  disc = _dis_of(deg[:, :])
  dist = _dis_of(degt[:, :])
  scale = -dist / (degt[:, :] + 2.0)
  lcd = lc[:, :] * disc
  for b in range(B):
    acc = jnp.dot(sa[b] * lcd, t0[b], preferred_element_type=jnp.float32)
    t1[b] = scale * acc


def _k2_body(lc, deg, degt, t1f, t0t, t1t, w, bias, out):
  disc = _dis_of(deg[:, :])
  dist = _dis_of(degt[:, :])
  scale = -2.0 * dist / (degt[:, :] + 2.0)
  mc = lc[:, :] * disc
  for b in range(B):
    prop = scale * jnp.dot(mc, t1f[b], preferred_element_type=jnp.float32)
    t2 = prop - t0t[b]
    out[b] = (jnp.dot(t0t[b], w[0], preferred_element_type=jnp.float32)
              + jnp.dot(t1t[b], w[1], preferred_element_type=jnp.float32)
              + jnp.dot(t2, w[2], preferred_element_type=jnp.float32)
              + bias[:, :])


def kernel(x, edge_index, spatial_attention, weight, bias):
  row = edge_index[0]
  col = edge_index[1]
  lc16, degp = _build_lcount(row, col)
  lc = lc16.reshape(N, N)

  t0 = pl.pallas_call(
      _k0_body,
      grid=(TGRID,),
      in_specs=[
          pl.BlockSpec((B, TR, TR), lambda t: (0, t, t)),
          pl.BlockSpec((B, TR, F), lambda t: (0, t, 0)),
      ],
      out_specs=pl.BlockSpec((B, TR, F), lambda t: (0, t, 0)),
      out_shape=jax.ShapeDtypeStruct((B, N, F), jnp.float32),
  )(spatial_attention, x)

  deg = degp.reshape(1, N)
  degt = degp.reshape(N, 1)

  t1 = pl.pallas_call(
      _k1_body,
      grid=(TGRID,),
      in_specs=[
          pl.BlockSpec((B, TR, N), lambda t: (0, t, 0)),
          pl.BlockSpec((TR, N), lambda t: (t, 0)),
          pl.BlockSpec((1, N), lambda t: (0, 0)),
          pl.BlockSpec((TR, 1), lambda t: (t, 0)),
          pl.BlockSpec((B, N, F), lambda t: (0, 0, 0)),
      ],
      out_specs=pl.BlockSpec((B, TR, F), lambda t: (0, t, 0)),
      out_shape=jax.ShapeDtypeStruct((B, N, F), jnp.float32),
  )(spatial_attention, lc, deg, degt, t0)

  out = pl.pallas_call(
      _k2_body,
      grid=(TGRID,),
      in_specs=[
          pl.BlockSpec((TR, N), lambda t: (t, 0)),
          pl.BlockSpec((1, N), lambda t: (0, 0)),
          pl.BlockSpec((TR, 1), lambda t: (t, 0)),
          pl.BlockSpec((B, N, F), lambda t: (0, 0, 0)),
          pl.BlockSpec((B, TR, F), lambda t: (0, t, 0)),
          pl.BlockSpec((B, TR, F), lambda t: (0, t, 0)),
          pl.BlockSpec((3, F, F), lambda t: (0, 0, 0)),
          pl.BlockSpec((1, F), lambda t: (0, 0)),
      ],
      out_specs=pl.BlockSpec((B, TR, F), lambda t: (0, t, 0)),
      out_shape=jax.ShapeDtypeStruct((B, N, F), jnp.float32),
  )(lc, deg, degt, t1, t0, t1, weight, bias.reshape(1, F))

  return out
